# baseline (speedup 1.0000x reference)
import functools

import jax
import jax.numpy as jnp
from jax import lax
from jax.experimental import pallas as pl
from jax.experimental.pallas import tpu as pltpu

_BITS_HI = 0x40000000
_MAX_ITERS = 30
_SPLITS = 4


def _masked_count_sum(x, tau):
    n = x.shape[1]
    step = n // _SPLITS
    cnts, sums = [], []
    for k in range(_SPLITS):
        lo = k * step
        hi = n if k == _SPLITS - 1 else (k + 1) * step
        xk = x[:, lo:hi]
        ge = xk >= tau
        cnts.append(jnp.sum(jnp.where(ge, 1.0, 0.0), axis=1, keepdims=True))
        sums.append(jnp.sum(jnp.where(ge, xk, 0.0), axis=1, keepdims=True))
    return sum(cnts), sum(sums)


def _split_reduce(x, fn):
    n = x.shape[1]
    step = n // _SPLITS
    parts = []
    for k in range(_SPLITS):
        lo = k * step
        hi = n if k == _SPLITS - 1 else (k + 1) * step
        parts.append(fn(x[:, lo:hi]))
    return functools.reduce(jnp.maximum, parts) if fn is _rowmax else sum(parts)


def _rowmax(x):
    return jnp.max(x, axis=1, keepdims=True)


def _rowsum(x):
    return jnp.sum(x, axis=1, keepdims=True)


def _raps_body(qhat_ref, logits_ref, pen_ref, mask_ref, sizes_ref, e_ref):
    l = logits_ref[...]
    m = _split_reduce(l, _rowmax)
    e = jnp.exp(l - m)
    e_ref[...] = e
    s = _split_reduce(e, _rowsum)

    pen_row = pen_ref[...]
    v = pen_row.shape[1]
    lam = pen_row[:, v - 1:v]
    kreg = jnp.sum((pen_row == 0.0).astype(jnp.float32), axis=1, keepdims=True)
    qhat_s = qhat_ref[0] * s
    lam_s = lam * s

    br = l.shape[0]
    lo0 = jnp.zeros((br, 1), jnp.int32)
    hi0 = jnp.full((br, 1), _BITS_HI, jnp.int32)
    cnt_hi0 = jnp.zeros((br, 1), jnp.float32)

    def body(_, carry):
        lo, hi, cnt_hi = carry
        mid = (lo + hi) >> 1
        tau = lax.bitcast_convert_type(mid, jnp.float32)
        cnt, ssum = _masked_count_sum(e_ref[...], tau)
        g = ssum + lam_s * jnp.maximum(cnt - kreg, 0.0)
        ok = g <= qhat_s
        lo = jnp.where(ok, lo, mid)
        hi = jnp.where(ok, mid, hi)
        cnt_hi = jnp.where(ok, cnt, cnt_hi)
        return lo, hi, cnt_hi

    _, hi, cnt_hi = lax.fori_loop(0, _MAX_ITERS, body, (lo0, hi0, cnt_hi0))

    tau_star = lax.bitcast_convert_type(hi, jnp.float32)
    ee = e_ref[...]
    thresh = _split_reduce(jnp.where(ee < tau_star, ee, -1.0), _rowmax)
    mask_ref[...] = ee >= thresh
    sizes = cnt_hi.astype(jnp.int32) + 1
    sizes_ref[...] = jnp.minimum(sizes, jnp.int32(v))


@jax.jit
def _raps_call(logits, penalties, qhat_arr):
    b, v = logits.shape
    br = 8
    grid = (b // br,)
    mask, sizes = pl.pallas_call(
        _raps_body,
        grid=grid,
        in_specs=[
            pl.BlockSpec(memory_space=pltpu.SMEM),
            pl.BlockSpec((br, v), lambda i: (i, 0)),
            pl.BlockSpec((1, v), lambda i: (0, 0)),
        ],
        out_specs=[
            pl.BlockSpec((br, v), lambda i: (i, 0)),
            pl.BlockSpec((br, 1), lambda i: (i, 0)),
        ],
        out_shape=[
            jax.ShapeDtypeStruct((b, v), jnp.bool_),
            jax.ShapeDtypeStruct((b, 1), jnp.int32),
        ],
        scratch_shapes=[pltpu.VMEM((br, v), jnp.float32)],
        compiler_params=pltpu.CompilerParams(
            dimension_semantics=("arbitrary",),
        ),
    )(qhat_arr, logits, penalties)
    return mask, sizes


def kernel(logits, penalties, Qhat):
    b, v = logits.shape
    qhat_arr = jnp.asarray(Qhat, jnp.float32).reshape(1)
    mask, sizes = _raps_call(logits, penalties, qhat_arr)
    return (logits, mask, sizes.reshape(b))


# speedup vs baseline: 1.3844x; 1.3844x over previous
import functools

import jax
import jax.numpy as jnp
from jax import lax
from jax.experimental import pallas as pl
from jax.experimental.pallas import tpu as pltpu

_BITS_HI = 0x40000000
_MAX_ITERS = 30
_SPLITS = 4


def _masked_count_sum(x, tau):
    ge = x >= tau
    cnt = jnp.sum(jnp.where(ge, 1.0, 0.0), axis=1, keepdims=True)
    ssum = jnp.sum(jnp.where(ge, x, 0.0), axis=1, keepdims=True)
    return cnt, ssum


def _rowmax(x):
    return jnp.max(x, axis=1, keepdims=True)


def _rowsum(x):
    return jnp.sum(x, axis=1, keepdims=True)


def _raps_body(qhat_ref, logits_ref, pen_ref, mask_ref, sizes_ref, e_ref):
    l = logits_ref[...]
    m = _rowmax(l)
    e = jnp.exp(l - m)
    e_ref[...] = e
    s = _rowsum(e)

    pen_row = pen_ref[...]
    v = pen_row.shape[1]
    lam = pen_row[:, v - 1:v]
    kreg = jnp.sum((pen_row == 0.0).astype(jnp.float32), axis=1, keepdims=True)
    qhat_s = qhat_ref[0] * s
    lam_s = lam * s

    br = l.shape[0]
    lo0 = jnp.zeros((br, 1), jnp.int32)
    hi0 = jnp.full((br, 1), _BITS_HI, jnp.int32)
    cnt_hi0 = jnp.zeros((br, 1), jnp.float32)

    def body(_, carry):
        lo, hi, cnt_hi = carry
        mid = (lo + hi) >> 1
        tau = lax.bitcast_convert_type(mid, jnp.float32)
        cnt, ssum = _masked_count_sum(e_ref[...], tau)
        g = ssum + lam_s * jnp.maximum(cnt - kreg, 0.0)
        ok = g <= qhat_s
        lo = jnp.where(ok, lo, mid)
        hi = jnp.where(ok, mid, hi)
        cnt_hi = jnp.where(ok, cnt, cnt_hi)
        return lo, hi, cnt_hi

    _, hi, cnt_hi = lax.fori_loop(0, _MAX_ITERS, body, (lo0, hi0, cnt_hi0))

    tau_star = lax.bitcast_convert_type(hi, jnp.float32)
    ee = e_ref[...]
    thresh = _rowmax(jnp.where(ee < tau_star, ee, -1.0))
    mask_ref[...] = ee >= thresh
    sizes = cnt_hi.astype(jnp.int32) + 1
    sizes_ref[...] = jnp.minimum(sizes, jnp.int32(v))


@jax.jit
def _raps_call(logits, penalties, qhat_arr):
    b, v = logits.shape
    br = 8
    grid = (b // br,)
    mask, sizes = pl.pallas_call(
        _raps_body,
        grid=grid,
        in_specs=[
            pl.BlockSpec(memory_space=pltpu.SMEM),
            pl.BlockSpec((br, v), lambda i: (i, 0)),
            pl.BlockSpec((1, v), lambda i: (0, 0)),
        ],
        out_specs=[
            pl.BlockSpec((br, v), lambda i: (i, 0)),
            pl.BlockSpec((br, 1), lambda i: (i, 0)),
        ],
        out_shape=[
            jax.ShapeDtypeStruct((b, v), jnp.bool_),
            jax.ShapeDtypeStruct((b, 1), jnp.int32),
        ],
        scratch_shapes=[pltpu.VMEM((br, v), jnp.float32)],
        compiler_params=pltpu.CompilerParams(
            dimension_semantics=("arbitrary",),
        ),
    )(qhat_arr, logits, penalties)
    return mask, sizes


def kernel(logits, penalties, Qhat):
    b, v = logits.shape
    qhat_arr = jnp.asarray(Qhat, jnp.float32).reshape(1)
    mask, sizes = _raps_call(logits, penalties, qhat_arr)
    return (logits, mask, sizes.reshape(b))


# speedup vs baseline: 1.5692x; 1.1335x over previous
import functools

import jax
import jax.numpy as jnp
from jax import lax
from jax.experimental import pallas as pl
from jax.experimental.pallas import tpu as pltpu

_BITS_HI = 0x40000000
_MAX_ITERS = 30
_SPLITS = 4


def _masked_count_sum(x, tau):
    ge = x >= tau
    cnt = jnp.sum(jnp.where(ge, 1.0, 0.0), axis=1, keepdims=True)
    ssum = jnp.sum(jnp.where(ge, x, 0.0), axis=1, keepdims=True)
    return cnt, ssum


def _rowmax(x):
    return jnp.max(x, axis=1, keepdims=True)


def _rowsum(x):
    return jnp.sum(x, axis=1, keepdims=True)


def _raps_body(qhat_ref, logits_ref, pen_ref, mask_ref, sizes_ref, e_ref):
    l = logits_ref[...]
    m = _rowmax(l)
    e = jnp.exp(l - m)
    e_ref[...] = e
    s = _rowsum(e)

    pen_row = pen_ref[...]
    v = pen_row.shape[1]
    lam = pen_row[:, v - 1:v]
    kreg = jnp.sum((pen_row == 0.0).astype(jnp.float32), axis=1, keepdims=True)
    qhat_s = qhat_ref[0] * s
    lam_s = lam * s

    br = l.shape[0]
    lo0 = jnp.zeros((br, 1), jnp.int32)
    hi0 = jnp.full((br, 1), _BITS_HI, jnp.int32)
    cnt_lo0 = jnp.full((br, 1), jnp.float32(v))
    cnt_hi0 = jnp.zeros((br, 1), jnp.float32)

    def cond(carry):
        it, lo, hi, cnt_lo, cnt_hi = carry
        return jnp.logical_and(it < _MAX_ITERS,
                               jnp.any(cnt_lo - cnt_hi > 1.0))

    def body(carry):
        it, lo, hi, cnt_lo, cnt_hi = carry
        mid = (lo + hi) >> 1
        tau = lax.bitcast_convert_type(mid, jnp.float32)
        cnt, ssum = _masked_count_sum(e_ref[...], tau)
        g = ssum + lam_s * jnp.maximum(cnt - kreg, 0.0)
        ok = g <= qhat_s
        lo = jnp.where(ok, lo, mid)
        hi = jnp.where(ok, mid, hi)
        cnt_lo = jnp.where(ok, cnt_lo, cnt)
        cnt_hi = jnp.where(ok, cnt, cnt_hi)
        return it + 1, lo, hi, cnt_lo, cnt_hi

    _, _, hi, _, cnt_hi = lax.while_loop(
        cond, body, (jnp.int32(0), lo0, hi0, cnt_lo0, cnt_hi0))

    tau_star = lax.bitcast_convert_type(hi, jnp.float32)
    ee = e_ref[...]
    thresh = _rowmax(jnp.where(ee < tau_star, ee, -1.0))
    mask_ref[...] = ee >= thresh
    sizes = cnt_hi.astype(jnp.int32) + 1
    sizes_ref[...] = jnp.minimum(sizes, jnp.int32(v))


@jax.jit
def _raps_call(logits, penalties, qhat_arr):
    b, v = logits.shape
    br = 8
    grid = (b // br,)
    mask, sizes = pl.pallas_call(
        _raps_body,
        grid=grid,
        in_specs=[
            pl.BlockSpec(memory_space=pltpu.SMEM),
            pl.BlockSpec((br, v), lambda i: (i, 0)),
            pl.BlockSpec((1, v), lambda i: (0, 0)),
        ],
        out_specs=[
            pl.BlockSpec((br, v), lambda i: (i, 0)),
            pl.BlockSpec((br, 1), lambda i: (i, 0)),
        ],
        out_shape=[
            jax.ShapeDtypeStruct((b, v), jnp.bool_),
            jax.ShapeDtypeStruct((b, 1), jnp.int32),
        ],
        scratch_shapes=[pltpu.VMEM((br, v), jnp.float32)],
        compiler_params=pltpu.CompilerParams(
            dimension_semantics=("arbitrary",),
        ),
    )(qhat_arr, logits, penalties)
    return mask, sizes


def kernel(logits, penalties, Qhat):
    b, v = logits.shape
    qhat_arr = jnp.asarray(Qhat, jnp.float32).reshape(1)
    mask, sizes = _raps_call(logits, penalties, qhat_arr)
    return (logits, mask, sizes.reshape(b))


# speedup vs baseline: 1.8638x; 1.1878x over previous
import functools

import jax
import jax.numpy as jnp
from jax import lax
from jax.experimental import pallas as pl
from jax.experimental.pallas import tpu as pltpu

_BITS_HI = 0x40000000
_MAX_ITERS = 30
_SPLITS = 4


def _bounds(n):
    step = ((n // _SPLITS) // 128) * 128
    return [k * step for k in range(_SPLITS)] + [n]


def _masked_count_sum(x, tau):
    bs = _bounds(x.shape[1])
    cnt = ssum = 0.0
    for k in range(_SPLITS):
        xk = x[:, bs[k]:bs[k + 1]]
        ge = xk >= tau
        cnt = cnt + jnp.sum(jnp.where(ge, 1.0, 0.0), axis=1, keepdims=True)
        ssum = ssum + jnp.sum(jnp.where(ge, xk, 0.0), axis=1, keepdims=True)
    return cnt, ssum


def _rowmax(x):
    bs = _bounds(x.shape[1])
    parts = [jnp.max(x[:, bs[k]:bs[k + 1]], axis=1, keepdims=True)
             for k in range(_SPLITS)]
    return functools.reduce(jnp.maximum, parts)


def _rowsum(x):
    bs = _bounds(x.shape[1])
    return sum(jnp.sum(x[:, bs[k]:bs[k + 1]], axis=1, keepdims=True)
               for k in range(_SPLITS))


def _raps_body(qhat_ref, logits_ref, pen_ref, mask_ref, sizes_ref, e_ref):
    l = logits_ref[...]
    m = _rowmax(l)
    e = jnp.exp(l - m)
    e_ref[...] = e
    s = _rowsum(e)

    pen_row = pen_ref[...]
    v = pen_row.shape[1]
    lam = pen_row[:, v - 1:v]
    kreg = jnp.sum((pen_row == 0.0).astype(jnp.float32), axis=1, keepdims=True)
    qhat_s = qhat_ref[0] * s
    lam_s = lam * s

    br = l.shape[0]
    lo0 = jnp.zeros((br, 1), jnp.int32)
    hi0 = jnp.full((br, 1), _BITS_HI, jnp.int32)
    cnt_lo0 = jnp.full((br, 1), jnp.float32(v))
    cnt_hi0 = jnp.zeros((br, 1), jnp.float32)

    def cond(carry):
        it, lo, hi, cnt_lo, cnt_hi = carry
        return jnp.logical_and(it < _MAX_ITERS,
                               jnp.any(cnt_lo - cnt_hi > 1.0))

    def body(carry):
        it, lo, hi, cnt_lo, cnt_hi = carry
        mid = (lo + hi) >> 1
        tau = lax.bitcast_convert_type(mid, jnp.float32)
        cnt, ssum = _masked_count_sum(e_ref[...], tau)
        g = ssum + lam_s * jnp.maximum(cnt - kreg, 0.0)
        ok = g <= qhat_s
        lo = jnp.where(ok, lo, mid)
        hi = jnp.where(ok, mid, hi)
        cnt_lo = jnp.where(ok, cnt_lo, cnt)
        cnt_hi = jnp.where(ok, cnt, cnt_hi)
        return it + 1, lo, hi, cnt_lo, cnt_hi

    _, _, hi, _, cnt_hi = lax.while_loop(
        cond, body, (jnp.int32(0), lo0, hi0, cnt_lo0, cnt_hi0))

    tau_star = lax.bitcast_convert_type(hi, jnp.float32)
    ee = e_ref[...]
    thresh = _rowmax(jnp.where(ee < tau_star, ee, -1.0))
    mask_ref[...] = ee >= thresh
    sizes = cnt_hi.astype(jnp.int32) + 1
    sizes_ref[...] = jnp.minimum(sizes, jnp.int32(v))


@jax.jit
def _raps_call(logits, penalties, qhat_arr):
    b, v = logits.shape
    br = 8
    grid = (b // br,)
    mask, sizes = pl.pallas_call(
        _raps_body,
        grid=grid,
        in_specs=[
            pl.BlockSpec(memory_space=pltpu.SMEM),
            pl.BlockSpec((br, v), lambda i: (i, 0)),
            pl.BlockSpec((1, v), lambda i: (0, 0)),
        ],
        out_specs=[
            pl.BlockSpec((br, v), lambda i: (i, 0)),
            pl.BlockSpec((br, 1), lambda i: (i, 0)),
        ],
        out_shape=[
            jax.ShapeDtypeStruct((b, v), jnp.bool_),
            jax.ShapeDtypeStruct((b, 1), jnp.int32),
        ],
        scratch_shapes=[pltpu.VMEM((br, v), jnp.float32)],
        compiler_params=pltpu.CompilerParams(
            dimension_semantics=("arbitrary",),
        ),
    )(qhat_arr, logits, penalties)
    return mask, sizes


def kernel(logits, penalties, Qhat):
    b, v = logits.shape
    qhat_arr = jnp.asarray(Qhat, jnp.float32).reshape(1)
    mask, sizes = _raps_call(logits, penalties, qhat_arr)
    return (logits, mask, sizes.reshape(b))
